# baseline (speedup 1.0000x reference)
import functools

import jax
import jax.numpy as jnp
from jax import lax
from jax.experimental import pallas as pl
from jax.experimental.pallas import tpu as pltpu
from jax.experimental.pallas import tpu_sc as plsc

B = 16384
D = 128
NC, NS = 2, 16
NW = NC * NS
BPW = B // NW
CH = 64
NCHUNK = BPW // CH
GB = (0.0, 0.0, 1.0, 0.5, 0.5)

_f32 = jnp.float32
_i32 = jnp.int32

_mesh = plsc.VectorSubcoreMesh(core_axis_name="c", subcore_axis_name="s",
                               num_cores=NC, num_subcores=NS)

_SC_OUT = (
    [jax.ShapeDtypeStruct((B,), _f32) for _ in range(5)]
    + [jax.ShapeDtypeStruct((B,), _f32),
       jax.ShapeDtypeStruct((B,), _f32),
       jax.ShapeDtypeStruct((B, 5), _f32)]
)

_SC_SCRATCH = [
    pltpu.VMEM((BPW,), _i32),
    pltpu.VMEM((BPW,), _i32),
    pltpu.VMEM((CH, D), _f32),
    pltpu.VMEM((CH, D), _f32),
    [pltpu.VMEM((BPW,), _f32) for _ in range(5)],
    [pltpu.VMEM((BPW,), _f32) for _ in range(5)],
    pltpu.VMEM((BPW,), _f32),
    pltpu.VMEM((BPW,), _f32),
    pltpu.VMEM((BPW, 5), _f32),
    [pltpu.VMEM((BPW,), _f32) for _ in range(5)],
    pltpu.SemaphoreType.DMA,
]


@functools.partial(pl.kernel, out_type=_SC_OUT, mesh=_mesh,
                   scratch_types=_SC_SCRATCH,
                   compiler_params=pltpu.CompilerParams(
                       needs_layout_passes=False,
                       use_tc_tiling_on_sc=False))
def _sc_gather_dot(users, items, dist, price, refpt,
                   uB_a, iB_a, uE_a, iE_a,
                   uB_b, iB_b, uE_b, iE_b,
                   uB_l, iB_l, uE_l, iE_l,
                   uB_g, iB_g, uE_g, iE_g,
                   uB_d, iB_d, uE_d, iE_d,
                   alpha_o, beta_o, lamda_o, gamma_o, delta_o,
                   price_o, ref_o, dist_o,
                   uidx, iidx, ubuf, ibuf, ubias, ibias,
                   pricebuf, refbuf, distbuf, dots, sem):
    wid = lax.axis_index("s") * NC + lax.axis_index("c")
    base = wid * BPW

    pltpu.sync_copy(users.at[pl.ds(base, BPW)], uidx)
    pltpu.sync_copy(items.at[pl.ds(base, BPW)], iidx)

    pltpu.async_copy(price.at[iidx], pricebuf, sem).wait()
    pltpu.async_copy(refpt.at[uidx], refbuf, sem).wait()
    pltpu.async_copy(dist.at[iidx], distbuf, sem).wait()
    ub_tabs = (uB_a, uB_b, uB_l, uB_g, uB_d)
    ib_tabs = (iB_a, iB_b, iB_l, iB_g, iB_d)
    for f in range(5):
        pltpu.async_copy(ub_tabs[f].at[uidx], ubias[f], sem).wait()
        pltpu.async_copy(ib_tabs[f].at[iidx], ibias[f], sem).wait()

    ue_tabs = (uE_a, uE_b, uE_l, uE_g, uE_d)
    ie_tabs = (iE_a, iE_b, iE_l, iE_g, iE_d)
    iota16 = lax.iota(_i32, 16)

    for f in range(5):
        uE, iE = ue_tabs[f], ie_tabs[f]

        def chunk_body(ci, carry, uE=uE, iE=iE, f=f):
            pltpu.async_copy(uE.at[uidx.at[pl.ds(ci * CH, CH)]], ubuf,
                             sem).wait()
            pltpu.async_copy(iE.at[iidx.at[pl.ds(ci * CH, CH)]], ibuf,
                             sem).wait()

            def group_body(g, carry2, f=f):
                rows = iota16 + g * 16

                def dbody(dd, acc):
                    for k in range(8):
                        cols = jnp.full((16,), dd * 8 + k, _i32)
                        u = plsc.load_gather(ubuf, [rows, cols])
                        v = plsc.load_gather(ibuf, [rows, cols])
                        acc = acc + u * v
                    return acc

                tot = lax.fori_loop(0, D // 8, dbody, jnp.zeros((16,), _f32))
                start = ci * CH + g * 16
                tot = tot + ubias[f][pl.ds(start, 16)] + ibias[f][pl.ds(start, 16)]
                if GB[f] != 0.0:
                    tot = tot + GB[f]
                dots[f][pl.ds(start, 16)] = tot
                return carry2

            lax.fori_loop(0, CH // 16, group_body, 0)
            return carry

        lax.fori_loop(0, NCHUNK, chunk_body, 0)

    outs = (alpha_o, beta_o, lamda_o, gamma_o, delta_o)
    for f in range(5):
        pltpu.sync_copy(dots[f], outs[f].at[pl.ds(base, BPW)])
    pltpu.sync_copy(pricebuf, price_o.at[pl.ds(base, BPW)])
    pltpu.sync_copy(refbuf, ref_o.at[pl.ds(base, BPW)])
    pltpu.sync_copy(distbuf, dist_o.at[pl.ds(base, BPW)])


def _pt_body(al_r, be_r, la_r, ga_r, de_r, pr_r, rf_r,
             p0_r, p1_r, p2_r, p3_r, p4_r, o_r):
    alpha = al_r[...]
    beta = be_r[...]
    lamda = la_r[...]
    gamma = ga_r[...]
    delta = de_r[...]
    price = pr_r[...]
    refv = rf_r[...]
    ps = (p0_r[...], p1_r[...], p2_r[...], p3_r[...], p4_r[...])

    acc = jnp.zeros_like(alpha)
    for r in range(5):
        x = jnp.tanh((r + 1.0) - refv)
        pos = (x > 0).astype(_f32)
        neg = 1.0 - pos
        x_ = price * jnp.abs(x) + 1e-8
        v_exp = alpha * pos + beta * neg
        v = x_ ** v_exp
        value = v * (pos - lamda * neg)
        p = ps[r]
        one_m_p = 1.0 - p
        w_g = (p ** gamma) / ((p ** gamma + one_m_p ** gamma) ** (1.0 / gamma))
        w_d = (p ** delta) / ((p ** delta + one_m_p ** delta) ** (1.0 / delta))
        weight = pos * w_g + neg * w_d
        acc = acc + weight * value
    o_r[...] = acc


_pt_call = pl.pallas_call(
    _pt_body, out_shape=jax.ShapeDtypeStruct((B // D, D), _f32))


def kernel(users, items, distribution, item_price, reference_point,
           uB_a, iB_a, uE_a, iE_a,
           uB_b, iB_b, uE_b, iE_b,
           uB_l, iB_l, uE_l, iE_l,
           uB_g, iB_g, uE_g, iE_g,
           uB_d, iB_d, uE_d, iE_d):
    users = users.astype(_i32)
    items = items.astype(_i32)
    flat = lambda t: t.reshape(-1)

    (alpha, beta, lamda, gamma, delta, price_g, ref_g, dist_g) = \
        _sc_gather_dot(users, items, distribution, item_price,
                       flat(reference_point),
                       flat(uB_a), flat(iB_a), uE_a, iE_a,
                       flat(uB_b), flat(iB_b), uE_b, iE_b,
                       flat(uB_l), flat(iB_l), uE_l, iE_l,
                       flat(uB_g), flat(iB_g), uE_g, iE_g,
                       flat(uB_d), flat(iB_d), uE_d, iE_d)

    r2 = lambda t: t.reshape(B // D, D)
    ps = [r2(dist_g[:, r]) for r in range(5)]
    out = _pt_call(r2(alpha), r2(beta), r2(lamda), r2(gamma), r2(delta),
                   r2(price_g), r2(ref_g), *ps)
    return out.reshape(B)


# speedup vs baseline: 1.8519x; 1.8519x over previous
import functools

import jax
import jax.numpy as jnp
from jax import lax
from jax.experimental import pallas as pl
from jax.experimental.pallas import tpu as pltpu
from jax.experimental.pallas import tpu_sc as plsc

B = 16384
D = 128
NC, NS = 2, 16
NW = NC * NS
BPW = B // NW
CH = 64
NCHUNK = BPW // CH
GB = (0.0, 0.0, 1.0, 0.5, 0.5)

_f32 = jnp.float32
_i32 = jnp.int32

_mesh = plsc.VectorSubcoreMesh(core_axis_name="c", subcore_axis_name="s",
                               num_cores=NC, num_subcores=NS)

_SC_OUT = (
    [jax.ShapeDtypeStruct((B,), _f32) for _ in range(5)]
    + [jax.ShapeDtypeStruct((B,), _f32),
       jax.ShapeDtypeStruct((B,), _f32),
       jax.ShapeDtypeStruct((B, 5), _f32)]
)

_SC_SCRATCH = [
    pltpu.VMEM((BPW,), _i32),
    pltpu.VMEM((BPW,), _i32),
    pltpu.VMEM((CH, D), _f32),
    pltpu.VMEM((CH, D), _f32),
    [pltpu.VMEM((BPW,), _f32) for _ in range(5)],
    [pltpu.VMEM((BPW,), _f32) for _ in range(5)],
    pltpu.VMEM((BPW,), _f32),
    pltpu.VMEM((BPW,), _f32),
    pltpu.VMEM((BPW, 5), _f32),
    [pltpu.VMEM((BPW,), _f32) for _ in range(5)],
    pltpu.VMEM((16, 17), _f32),
    pltpu.SemaphoreType.DMA,
]


@functools.partial(pl.kernel, out_type=_SC_OUT, mesh=_mesh,
                   scratch_types=_SC_SCRATCH,
                   compiler_params=pltpu.CompilerParams(
                       needs_layout_passes=False,
                       use_tc_tiling_on_sc=False))
def _sc_gather_dot(users, items, dist, price, refpt,
                   uB_a, iB_a, uE_a, iE_a,
                   uB_b, iB_b, uE_b, iE_b,
                   uB_l, iB_l, uE_l, iE_l,
                   uB_g, iB_g, uE_g, iE_g,
                   uB_d, iB_d, uE_d, iE_d,
                   alpha_o, beta_o, lamda_o, gamma_o, delta_o,
                   price_o, ref_o, dist_o,
                   uidx, iidx, ubuf, ibuf, ubias, ibias,
                   pricebuf, refbuf, distbuf, dots, tbuf, sem):
    wid = lax.axis_index("s") * NC + lax.axis_index("c")
    base = wid * BPW

    pltpu.sync_copy(users.at[pl.ds(base, BPW)], uidx)
    pltpu.sync_copy(items.at[pl.ds(base, BPW)], iidx)

    pltpu.async_copy(price.at[iidx], pricebuf, sem).wait()
    pltpu.async_copy(refpt.at[uidx], refbuf, sem).wait()
    pltpu.async_copy(dist.at[iidx], distbuf, sem).wait()
    ub_tabs = (uB_a, uB_b, uB_l, uB_g, uB_d)
    ib_tabs = (iB_a, iB_b, iB_l, iB_g, iB_d)
    for f in range(5):
        pltpu.async_copy(ub_tabs[f].at[uidx], ubias[f], sem).wait()
        pltpu.async_copy(ib_tabs[f].at[iidx], ibias[f], sem).wait()

    ue_tabs = (uE_a, uE_b, uE_l, uE_g, uE_d)
    ie_tabs = (iE_a, iE_b, iE_l, iE_g, iE_d)
    iota16 = lax.iota(_i32, 16)

    for f in range(5):
        uE, iE = ue_tabs[f], ie_tabs[f]

        def chunk_body(ci, carry, uE=uE, iE=iE, f=f):
            pltpu.async_copy(uE.at[uidx.at[pl.ds(ci * CH, CH)]], ubuf,
                             sem).wait()
            pltpu.async_copy(iE.at[iidx.at[pl.ds(ci * CH, CH)]], ibuf,
                             sem).wait()

            def group_body(g, carry2, f=f):
                for jj in range(16):
                    e = g * 16 + jj
                    prods = []
                    for c in range(D // 16):
                        u = ubuf[e, pl.ds(c * 16, 16)]
                        v = ibuf[e, pl.ds(c * 16, 16)]
                        prods.append(u * v)
                    while len(prods) > 1:
                        prods = [a + b for a, b in
                                 zip(prods[::2], prods[1::2])]
                    tbuf[jj, pl.ds(0, 16)] = prods[0]
                csums = [plsc.load_gather(
                    tbuf, [iota16, jnp.full((16,), c, _i32)])
                    for c in range(16)]
                while len(csums) > 1:
                    csums = [a + b for a, b in zip(csums[::2], csums[1::2])]
                tot = csums[0]
                start = ci * CH + g * 16
                tot = tot + ubias[f][pl.ds(start, 16)] + ibias[f][pl.ds(start, 16)]
                if GB[f] != 0.0:
                    tot = tot + GB[f]
                dots[f][pl.ds(start, 16)] = tot
                return carry2

            lax.fori_loop(0, CH // 16, group_body, 0)
            return carry

        lax.fori_loop(0, NCHUNK, chunk_body, 0)

    outs = (alpha_o, beta_o, lamda_o, gamma_o, delta_o)
    for f in range(5):
        pltpu.sync_copy(dots[f], outs[f].at[pl.ds(base, BPW)])
    pltpu.sync_copy(pricebuf, price_o.at[pl.ds(base, BPW)])
    pltpu.sync_copy(refbuf, ref_o.at[pl.ds(base, BPW)])
    pltpu.sync_copy(distbuf, dist_o.at[pl.ds(base, BPW)])


def _pt_body(al_r, be_r, la_r, ga_r, de_r, pr_r, rf_r,
             p0_r, p1_r, p2_r, p3_r, p4_r, o_r):
    alpha = al_r[...]
    beta = be_r[...]
    lamda = la_r[...]
    gamma = ga_r[...]
    delta = de_r[...]
    price = pr_r[...]
    refv = rf_r[...]
    ps = (p0_r[...], p1_r[...], p2_r[...], p3_r[...], p4_r[...])

    acc = jnp.zeros_like(alpha)
    for r in range(5):
        x = jnp.tanh((r + 1.0) - refv)
        pos = (x > 0).astype(_f32)
        neg = 1.0 - pos
        x_ = price * jnp.abs(x) + 1e-8
        v_exp = alpha * pos + beta * neg
        v = x_ ** v_exp
        value = v * (pos - lamda * neg)
        p = ps[r]
        one_m_p = 1.0 - p
        w_g = (p ** gamma) / ((p ** gamma + one_m_p ** gamma) ** (1.0 / gamma))
        w_d = (p ** delta) / ((p ** delta + one_m_p ** delta) ** (1.0 / delta))
        weight = pos * w_g + neg * w_d
        acc = acc + weight * value
    o_r[...] = acc


_pt_call = pl.pallas_call(
    _pt_body, out_shape=jax.ShapeDtypeStruct((B // D, D), _f32))


def kernel(users, items, distribution, item_price, reference_point,
           uB_a, iB_a, uE_a, iE_a,
           uB_b, iB_b, uE_b, iE_b,
           uB_l, iB_l, uE_l, iE_l,
           uB_g, iB_g, uE_g, iE_g,
           uB_d, iB_d, uE_d, iE_d):
    users = users.astype(_i32)
    items = items.astype(_i32)
    flat = lambda t: t.reshape(-1)

    (alpha, beta, lamda, gamma, delta, price_g, ref_g, dist_g) = \
        _sc_gather_dot(users, items, distribution, item_price,
                       flat(reference_point),
                       flat(uB_a), flat(iB_a), uE_a, iE_a,
                       flat(uB_b), flat(iB_b), uE_b, iE_b,
                       flat(uB_l), flat(iB_l), uE_l, iE_l,
                       flat(uB_g), flat(iB_g), uE_g, iE_g,
                       flat(uB_d), flat(iB_d), uE_d, iE_d)

    r2 = lambda t: t.reshape(B // D, D)
    ps = [r2(dist_g[:, r]) for r in range(5)]
    out = _pt_call(r2(alpha), r2(beta), r2(lamda), r2(gamma), r2(delta),
                   r2(price_g), r2(ref_g), *ps)
    return out.reshape(B)


# speedup vs baseline: 2.3670x; 1.2781x over previous
import functools

import jax
import jax.numpy as jnp
from jax import lax
from jax.experimental import pallas as pl
from jax.experimental.pallas import tpu as pltpu
from jax.experimental.pallas import tpu_sc as plsc

B = 16384
D = 128
NC, NS = 2, 16
NW = NC * NS
BPW = B // NW
CH = 128
NCHUNK = BPW // CH
GB = (0.0, 0.0, 1.0, 0.5, 0.5)

_f32 = jnp.float32
_i32 = jnp.int32

_mesh = plsc.VectorSubcoreMesh(core_axis_name="c", subcore_axis_name="s",
                               num_cores=NC, num_subcores=NS)

_SC_OUT = (
    [jax.ShapeDtypeStruct((B,), _f32) for _ in range(5)]
    + [jax.ShapeDtypeStruct((B,), _f32),
       jax.ShapeDtypeStruct((B,), _f32),
       jax.ShapeDtypeStruct((B, 5), _f32)]
)

_SC_SCRATCH = [
    pltpu.VMEM((BPW,), _i32),
    pltpu.VMEM((BPW,), _i32),
    [pltpu.VMEM((CH, D), _f32) for _ in range(2)],
    [pltpu.VMEM((CH, D), _f32) for _ in range(2)],
    [pltpu.VMEM((BPW,), _f32) for _ in range(5)],
    [pltpu.VMEM((BPW,), _f32) for _ in range(5)],
    pltpu.VMEM((BPW,), _f32),
    pltpu.VMEM((BPW,), _f32),
    pltpu.VMEM((BPW, 5), _f32),
    [pltpu.VMEM((BPW,), _f32) for _ in range(5)],
    pltpu.VMEM((16, 17), _f32),
    [pltpu.SemaphoreType.DMA for _ in range(2)],
    [pltpu.SemaphoreType.DMA for _ in range(2)],
    pltpu.SemaphoreType.DMA,
]


@functools.partial(pl.kernel, out_type=_SC_OUT, mesh=_mesh,
                   scratch_types=_SC_SCRATCH,
                   compiler_params=pltpu.CompilerParams(
                       needs_layout_passes=False,
                       use_tc_tiling_on_sc=False))
def _sc_gather_dot(users, items, dist, price, refpt,
                   uB_a, iB_a, uE_a, iE_a,
                   uB_b, iB_b, uE_b, iE_b,
                   uB_l, iB_l, uE_l, iE_l,
                   uB_g, iB_g, uE_g, iE_g,
                   uB_d, iB_d, uE_d, iE_d,
                   alpha_o, beta_o, lamda_o, gamma_o, delta_o,
                   price_o, ref_o, dist_o,
                   uidx, iidx, ubuf, ibuf, ubias, ibias,
                   pricebuf, refbuf, distbuf, dots, tbuf,
                   sem_u, sem_i, sem_misc):
    wid = lax.axis_index("s") * NC + lax.axis_index("c")
    base = wid * BPW

    pltpu.sync_copy(users.at[pl.ds(base, BPW)], uidx)
    pltpu.sync_copy(items.at[pl.ds(base, BPW)], iidx)

    ub_tabs = (uB_a, uB_b, uB_l, uB_g, uB_d)
    ib_tabs = (iB_a, iB_b, iB_l, iB_g, iB_d)
    ue_tabs = (uE_a, uE_b, uE_l, uE_g, uE_d)
    ie_tabs = (iE_a, iE_b, iE_l, iE_g, iE_d)
    iota16 = lax.iota(_i32, 16)

    misc = [pltpu.async_copy(price.at[iidx], pricebuf, sem_misc),
            pltpu.async_copy(refpt.at[uidx], refbuf, sem_misc),
            pltpu.async_copy(dist.at[iidx], distbuf, sem_misc)]
    for f in range(5):
        misc.append(pltpu.async_copy(ub_tabs[f].at[uidx], ubias[f], sem_misc))
        misc.append(pltpu.async_copy(ib_tabs[f].at[iidx], ibias[f], sem_misc))

    def issue(f, ci, slot):
        pltpu.async_copy(ue_tabs[f].at[uidx.at[pl.ds(ci * CH, CH)]],
                         ubuf[slot], sem_u[slot])
        pltpu.async_copy(ie_tabs[f].at[iidx.at[pl.ds(ci * CH, CH)]],
                         ibuf[slot], sem_i[slot])

    def wait(f, ci, slot):
        pltpu.make_async_copy(ue_tabs[f].at[uidx.at[pl.ds(ci * CH, CH)]],
                              ubuf[slot], sem_u[slot]).wait()
        pltpu.make_async_copy(ie_tabs[f].at[iidx.at[pl.ds(ci * CH, CH)]],
                              ibuf[slot], sem_i[slot]).wait()

    def compute_chunk(ci, ub, ib, f):
        def group_body(g, carry2, f=f):
            for jj in range(16):
                e = g * 16 + jj
                prods = []
                for c in range(D // 16):
                    u = ub[e, pl.ds(c * 16, 16)]
                    v = ib[e, pl.ds(c * 16, 16)]
                    prods.append(u * v)
                while len(prods) > 1:
                    prods = [a + b for a, b in zip(prods[::2], prods[1::2])]
                tbuf[jj, pl.ds(0, 16)] = prods[0]
            csums = [plsc.load_gather(
                tbuf, [iota16, jnp.full((16,), c, _i32)])
                for c in range(16)]
            while len(csums) > 1:
                csums = [a + b for a, b in zip(csums[::2], csums[1::2])]
            tot = csums[0]
            start = ci * CH + g * 16
            tot = tot + ubias[f][pl.ds(start, 16)] + ibias[f][pl.ds(start, 16)]
            if GB[f] != 0.0:
                tot = tot + GB[f]
            dots[f][pl.ds(start, 16)] = tot
            return carry2

        lax.fori_loop(0, CH // 16, group_body, 0)

    issue(0, 0, 0)
    for m in misc:
        m.wait()

    for f in range(5):
        def pair_body(s, carry, f=f):
            c0 = s * 2
            issue(f, c0 + 1, 1)
            wait(f, c0, 0)
            compute_chunk(c0, ubuf[0], ibuf[0], f)

            @pl.when(c0 + 2 < NCHUNK)
            def _():
                issue(f, c0 + 2, 0)

            wait(f, c0 + 1, 1)
            compute_chunk(c0 + 1, ubuf[1], ibuf[1], f)
            return carry

        lax.fori_loop(0, NCHUNK // 2, pair_body, 0)
        if f < 4:
            issue(f + 1, 0, 0)

    outs = (alpha_o, beta_o, lamda_o, gamma_o, delta_o)
    for f in range(5):
        pltpu.sync_copy(dots[f], outs[f].at[pl.ds(base, BPW)])
    pltpu.sync_copy(pricebuf, price_o.at[pl.ds(base, BPW)])
    pltpu.sync_copy(refbuf, ref_o.at[pl.ds(base, BPW)])
    pltpu.sync_copy(distbuf, dist_o.at[pl.ds(base, BPW)])


def _pt_body(al_r, be_r, la_r, ga_r, de_r, pr_r, rf_r,
             p0_r, p1_r, p2_r, p3_r, p4_r, o_r):
    alpha = al_r[...]
    beta = be_r[...]
    lamda = la_r[...]
    gamma = ga_r[...]
    delta = de_r[...]
    price = pr_r[...]
    refv = rf_r[...]
    ps = (p0_r[...], p1_r[...], p2_r[...], p3_r[...], p4_r[...])

    acc = jnp.zeros_like(alpha)
    for r in range(5):
        x = jnp.tanh((r + 1.0) - refv)
        pos = (x > 0).astype(_f32)
        neg = 1.0 - pos
        x_ = price * jnp.abs(x) + 1e-8
        v_exp = alpha * pos + beta * neg
        v = x_ ** v_exp
        value = v * (pos - lamda * neg)
        p = ps[r]
        one_m_p = 1.0 - p
        w_g = (p ** gamma) / ((p ** gamma + one_m_p ** gamma) ** (1.0 / gamma))
        w_d = (p ** delta) / ((p ** delta + one_m_p ** delta) ** (1.0 / delta))
        weight = pos * w_g + neg * w_d
        acc = acc + weight * value
    o_r[...] = acc


_pt_call = pl.pallas_call(
    _pt_body, out_shape=jax.ShapeDtypeStruct((B // D, D), _f32))


def kernel(users, items, distribution, item_price, reference_point,
           uB_a, iB_a, uE_a, iE_a,
           uB_b, iB_b, uE_b, iE_b,
           uB_l, iB_l, uE_l, iE_l,
           uB_g, iB_g, uE_g, iE_g,
           uB_d, iB_d, uE_d, iE_d):
    users = users.astype(_i32)
    items = items.astype(_i32)
    flat = lambda t: t.reshape(-1)

    (alpha, beta, lamda, gamma, delta, price_g, ref_g, dist_g) = \
        _sc_gather_dot(users, items, distribution, item_price,
                       flat(reference_point),
                       flat(uB_a), flat(iB_a), uE_a, iE_a,
                       flat(uB_b), flat(iB_b), uE_b, iE_b,
                       flat(uB_l), flat(iB_l), uE_l, iE_l,
                       flat(uB_g), flat(iB_g), uE_g, iE_g,
                       flat(uB_d), flat(iB_d), uE_d, iE_d)

    r2 = lambda t: t.reshape(B // D, D)
    ps = [r2(dist_g[:, r]) for r in range(5)]
    out = _pt_call(r2(alpha), r2(beta), r2(lamda), r2(gamma), r2(delta),
                   r2(price_g), r2(ref_g), *ps)
    return out.reshape(B)


# speedup vs baseline: 2.7292x; 1.1530x over previous
import functools

import jax
import jax.numpy as jnp
from jax import lax
from jax.experimental import pallas as pl
from jax.experimental.pallas import tpu as pltpu
from jax.experimental.pallas import tpu_sc as plsc

B = 16384
D = 128
NC, NS = 2, 16
NW = NC * NS
BPW = B // NW
CH = 128
NCHUNK = BPW // CH
GB = (0.0, 0.0, 1.0, 0.5, 0.5)

_f32 = jnp.float32
_i32 = jnp.int32

_mesh = plsc.VectorSubcoreMesh(core_axis_name="c", subcore_axis_name="s",
                               num_cores=NC, num_subcores=NS)

_SC_OUT = (
    [jax.ShapeDtypeStruct((B,), _f32) for _ in range(5)]
    + [jax.ShapeDtypeStruct((B,), _f32),
       jax.ShapeDtypeStruct((B,), _f32)]
    + [jax.ShapeDtypeStruct((B,), _f32) for _ in range(5)]
)

_SC_SCRATCH = [
    pltpu.VMEM((BPW,), _i32),
    pltpu.VMEM((BPW,), _i32),
    [pltpu.VMEM((CH, D), _f32) for _ in range(2)],
    [pltpu.VMEM((CH, D), _f32) for _ in range(2)],
    [pltpu.VMEM((BPW,), _f32) for _ in range(5)],
    [pltpu.VMEM((BPW,), _f32) for _ in range(5)],
    pltpu.VMEM((BPW,), _f32),
    pltpu.VMEM((BPW,), _f32),
    pltpu.VMEM((BPW, 5), _f32),
    [pltpu.VMEM((BPW,), _f32) for _ in range(5)],
    [pltpu.VMEM((BPW,), _f32) for _ in range(5)],
    pltpu.VMEM((16, 17), _f32),
    [pltpu.SemaphoreType.DMA for _ in range(2)],
    [pltpu.SemaphoreType.DMA for _ in range(2)],
    pltpu.SemaphoreType.DMA,
]


@functools.partial(pl.kernel, out_type=_SC_OUT, mesh=_mesh,
                   scratch_types=_SC_SCRATCH,
                   compiler_params=pltpu.CompilerParams(
                       needs_layout_passes=False,
                       use_tc_tiling_on_sc=False))
def _sc_gather_dot(users, items, dist, price, refpt,
                   uB_a, iB_a, uE_a, iE_a,
                   uB_b, iB_b, uE_b, iE_b,
                   uB_l, iB_l, uE_l, iE_l,
                   uB_g, iB_g, uE_g, iE_g,
                   uB_d, iB_d, uE_d, iE_d,
                   alpha_o, beta_o, lamda_o, gamma_o, delta_o,
                   price_o, ref_o, d0_o, d1_o, d2_o, d3_o, d4_o,
                   uidx, iidx, ubuf, ibuf, ubias, ibias,
                   pricebuf, refbuf, distbuf, distT, dots, tbuf,
                   sem_u, sem_i, sem_misc):
    wid = lax.axis_index("s") * NC + lax.axis_index("c")
    base = wid * BPW

    pltpu.sync_copy(users.at[pl.ds(base, BPW)], uidx)
    pltpu.sync_copy(items.at[pl.ds(base, BPW)], iidx)

    ub_tabs = (uB_a, uB_b, uB_l, uB_g, uB_d)
    ib_tabs = (iB_a, iB_b, iB_l, iB_g, iB_d)
    ue_tabs = (uE_a, uE_b, uE_l, uE_g, uE_d)
    ie_tabs = (iE_a, iE_b, iE_l, iE_g, iE_d)
    iota16 = lax.iota(_i32, 16)

    misc = [pltpu.async_copy(price.at[iidx], pricebuf, sem_misc),
            pltpu.async_copy(refpt.at[uidx], refbuf, sem_misc),
            pltpu.async_copy(dist.at[iidx], distbuf, sem_misc)]
    for f in range(5):
        misc.append(pltpu.async_copy(ub_tabs[f].at[uidx], ubias[f], sem_misc))
        misc.append(pltpu.async_copy(ib_tabs[f].at[iidx], ibias[f], sem_misc))

    def issue(f, ci, slot):
        pltpu.async_copy(ue_tabs[f].at[uidx.at[pl.ds(ci * CH, CH)]],
                         ubuf[slot], sem_u[slot])
        pltpu.async_copy(ie_tabs[f].at[iidx.at[pl.ds(ci * CH, CH)]],
                         ibuf[slot], sem_i[slot])

    def wait(f, ci, slot):
        pltpu.make_async_copy(ue_tabs[f].at[uidx.at[pl.ds(ci * CH, CH)]],
                              ubuf[slot], sem_u[slot]).wait()
        pltpu.make_async_copy(ie_tabs[f].at[iidx.at[pl.ds(ci * CH, CH)]],
                              ibuf[slot], sem_i[slot]).wait()

    def compute_chunk(ci, ub, ib, f):
        def group_body(g, carry2, f=f):
            for jj in range(16):
                e = g * 16 + jj
                prods = []
                for c in range(D // 16):
                    u = ub[e, pl.ds(c * 16, 16)]
                    v = ib[e, pl.ds(c * 16, 16)]
                    prods.append(u * v)
                while len(prods) > 1:
                    prods = [a + b for a, b in zip(prods[::2], prods[1::2])]
                tbuf[jj, pl.ds(0, 16)] = prods[0]
            csums = [plsc.load_gather(
                tbuf, [iota16, jnp.full((16,), c, _i32)])
                for c in range(16)]
            while len(csums) > 1:
                csums = [a + b for a, b in zip(csums[::2], csums[1::2])]
            tot = csums[0]
            start = ci * CH + g * 16
            tot = (tot + ubias[f][pl.ds(start, 16)]
                   + ibias[f][pl.ds(start, 16)])
            if GB[f] != 0.0:
                tot = tot + GB[f]
            dots[f][pl.ds(start, 16)] = tot
            return carry2

        lax.fori_loop(0, CH // 16, group_body, 0)

    issue(0, 0, 0)
    for m in misc:
        m.wait()

    for f in range(5):
        def pair_body(s, carry, f=f):
            c0 = s * 2
            issue(f, c0 + 1, 1)
            wait(f, c0, 0)
            compute_chunk(c0, ubuf[0], ibuf[0], f)

            @pl.when(c0 + 2 < NCHUNK)
            def _():
                issue(f, c0 + 2, 0)

            wait(f, c0 + 1, 1)
            compute_chunk(c0 + 1, ubuf[1], ibuf[1], f)
            return carry

        lax.fori_loop(0, NCHUNK // 2, pair_body, 0)
        if f < 4:
            issue(f + 1, 0, 0)

    def trans_body(g, carry):
        ridx = iota16 + g * 16
        for r in range(5):
            distT[r][pl.ds(g * 16, 16)] = plsc.load_gather(
                distbuf, [ridx, jnp.full((16,), r, _i32)])
        return carry

    lax.fori_loop(0, BPW // 16, trans_body, 0)

    outs = (alpha_o, beta_o, lamda_o, gamma_o, delta_o)
    for f in range(5):
        pltpu.sync_copy(dots[f], outs[f].at[pl.ds(base, BPW)])
    pltpu.sync_copy(pricebuf, price_o.at[pl.ds(base, BPW)])
    pltpu.sync_copy(refbuf, ref_o.at[pl.ds(base, BPW)])
    d_outs = (d0_o, d1_o, d2_o, d3_o, d4_o)
    for r in range(5):
        pltpu.sync_copy(distT[r], d_outs[r].at[pl.ds(base, BPW)])


def _pt_body(al_r, be_r, la_r, ga_r, de_r, pr_r, rf_r,
             p0_r, p1_r, p2_r, p3_r, p4_r, o_r):
    alpha = al_r[...]
    beta = be_r[...]
    lamda = la_r[...]
    gamma = ga_r[...]
    delta = de_r[...]
    price = pr_r[...]
    refv = rf_r[...]
    ps = (p0_r[...], p1_r[...], p2_r[...], p3_r[...], p4_r[...])

    acc = jnp.zeros_like(alpha)
    for r in range(5):
        x = jnp.tanh((r + 1.0) - refv)
        pos = (x > 0).astype(_f32)
        neg = 1.0 - pos
        x_ = price * jnp.abs(x) + 1e-8
        v_exp = alpha * pos + beta * neg
        v = x_ ** v_exp
        value = v * (pos - lamda * neg)
        p = ps[r]
        one_m_p = 1.0 - p
        w_g = (p ** gamma) / ((p ** gamma + one_m_p ** gamma) ** (1.0 / gamma))
        w_d = (p ** delta) / ((p ** delta + one_m_p ** delta) ** (1.0 / delta))
        weight = pos * w_g + neg * w_d
        acc = acc + weight * value
    o_r[...] = acc


_pt_call = pl.pallas_call(
    _pt_body, out_shape=jax.ShapeDtypeStruct((B,), _f32))


def kernel(users, items, distribution, item_price, reference_point,
           uB_a, iB_a, uE_a, iE_a,
           uB_b, iB_b, uE_b, iE_b,
           uB_l, iB_l, uE_l, iE_l,
           uB_g, iB_g, uE_g, iE_g,
           uB_d, iB_d, uE_d, iE_d):
    users = users.astype(_i32)
    items = items.astype(_i32)

    (alpha, beta, lamda, gamma, delta, price_g, ref_g,
     p0, p1, p2, p3, p4) = \
        _sc_gather_dot(users, items, distribution, item_price,
                       reference_point.reshape(-1),
                       uB_a.reshape(-1), iB_a.reshape(-1), uE_a, iE_a,
                       uB_b.reshape(-1), iB_b.reshape(-1), uE_b, iE_b,
                       uB_l.reshape(-1), iB_l.reshape(-1), uE_l, iE_l,
                       uB_g.reshape(-1), iB_g.reshape(-1), uE_g, iE_g,
                       uB_d.reshape(-1), iB_d.reshape(-1), uE_d, iE_d)

    return _pt_call(alpha, beta, lamda, gamma, delta,
                    price_g, ref_g, p0, p1, p2, p3, p4)


# speedup vs baseline: 3.4648x; 1.2695x over previous
import functools

import jax
import jax.numpy as jnp
from jax import lax
from jax.experimental import pallas as pl
from jax.experimental.pallas import tpu as pltpu
from jax.experimental.pallas import tpu_sc as plsc

B = 16384
D = 128
U_ROWS = 100000
I_ROWS = 100000
NC, NS = 2, 16
NW = NC * NS
BPW = B // NW
CH = 128
NCHUNK = BPW // CH
GB = (0.0, 0.0, 1.0, 0.5, 0.5)

_f32 = jnp.float32
_i32 = jnp.int32

_mesh = plsc.VectorSubcoreMesh(core_axis_name="c", subcore_axis_name="s",
                               num_cores=NC, num_subcores=NS)

_SC_OUT = (
    [jax.ShapeDtypeStruct((B,), _f32) for _ in range(5)]
    + [jax.ShapeDtypeStruct((B,), _f32),
       jax.ShapeDtypeStruct((B,), _f32)]
    + [jax.ShapeDtypeStruct((B,), _f32) for _ in range(5)]
)

_SC_SCRATCH = [
    pltpu.VMEM((BPW,), _i32),
    pltpu.VMEM((BPW,), _i32),
    [pltpu.VMEM((BPW,), _i32) for _ in range(5)],
    [pltpu.VMEM((BPW,), _i32) for _ in range(5)],
    [pltpu.VMEM((BPW,), _i32) for _ in range(4)],
    [pltpu.VMEM((CH, D), _f32) for _ in range(2)],
    [pltpu.VMEM((CH, D), _f32) for _ in range(2)],
    [pltpu.VMEM((BPW,), _f32) for _ in range(5)],
    [pltpu.VMEM((BPW,), _f32) for _ in range(5)],
    pltpu.VMEM((BPW,), _f32),
    pltpu.VMEM((BPW,), _f32),
    [pltpu.VMEM((BPW,), _f32) for _ in range(5)],
    [pltpu.VMEM((BPW,), _f32) for _ in range(5)],
    pltpu.VMEM((16, 17), _f32),
    [pltpu.SemaphoreType.DMA for _ in range(2)],
    [pltpu.SemaphoreType.DMA for _ in range(2)],
    pltpu.SemaphoreType.DMA,
]


@functools.partial(pl.kernel, out_type=_SC_OUT, mesh=_mesh,
                   scratch_types=_SC_SCRATCH,
                   compiler_params=pltpu.CompilerParams(
                       needs_layout_passes=False,
                       use_tc_tiling_on_sc=False))
def _sc_gather_dot(users, items, dist_cm, ubcat, ibcat,
                   uE_a, iE_a, uE_b, iE_b, uE_l, iE_l,
                   uE_g, iE_g, uE_d, iE_d,
                   alpha_o, beta_o, lamda_o, gamma_o, delta_o,
                   price_o, ref_o, d0_o, d1_o, d2_o, d3_o, d4_o,
                   uidx, iidx, uxidx, ixidx, dxidx, ubuf, ibuf, ubias, ibias,
                   pricebuf, refbuf, distT, dots, tbuf,
                   sem_u, sem_i, sem_misc):
    wid = lax.axis_index("s") * NC + lax.axis_index("c")
    base = wid * BPW

    pltpu.sync_copy(users.at[pl.ds(base, BPW)], uidx)
    pltpu.sync_copy(items.at[pl.ds(base, BPW)], iidx)

    ue_tabs = (uE_a, uE_b, uE_l, uE_g, uE_d)
    ie_tabs = (iE_a, iE_b, iE_l, iE_g, iE_d)
    iota16 = lax.iota(_i32, 16)

    def sidx_body(g, carry):
        sl = pl.ds(g * 16, 16)
        uv = uidx[sl]
        iv = iidx[sl]
        for k in range(5):
            uxidx[k][sl] = uv + ((k + 1) * U_ROWS)
            ixidx[k][sl] = iv + ((k + 1) * I_ROWS)
        for k in range(4):
            dxidx[k][sl] = iv + ((k + 1) * I_ROWS)
        return carry

    lax.fori_loop(0, BPW // 16, sidx_body, 0)

    misc = [pltpu.async_copy(ibcat.at[ixidx[4]], pricebuf, sem_misc),
            pltpu.async_copy(ubcat.at[uxidx[4]], refbuf, sem_misc),
            pltpu.async_copy(dist_cm.at[iidx], distT[0], sem_misc)]
    misc.append(pltpu.async_copy(ubcat.at[uidx], ubias[0], sem_misc))
    misc.append(pltpu.async_copy(ibcat.at[iidx], ibias[0], sem_misc))
    for f in range(1, 5):
        misc.append(pltpu.async_copy(ubcat.at[uxidx[f - 1]], ubias[f],
                                     sem_misc))
        misc.append(pltpu.async_copy(ibcat.at[ixidx[f - 1]], ibias[f],
                                     sem_misc))
    for r in range(1, 5):
        misc.append(pltpu.async_copy(dist_cm.at[dxidx[r - 1]], distT[r],
                                     sem_misc))

    def issue(f, ci, slot):
        pltpu.async_copy(ue_tabs[f].at[uidx.at[pl.ds(ci * CH, CH)]],
                         ubuf[slot], sem_u[slot])
        pltpu.async_copy(ie_tabs[f].at[iidx.at[pl.ds(ci * CH, CH)]],
                         ibuf[slot], sem_i[slot])

    def wait(f, ci, slot):
        pltpu.make_async_copy(ue_tabs[f].at[uidx.at[pl.ds(ci * CH, CH)]],
                              ubuf[slot], sem_u[slot]).wait()
        pltpu.make_async_copy(ie_tabs[f].at[iidx.at[pl.ds(ci * CH, CH)]],
                              ibuf[slot], sem_i[slot]).wait()

    def compute_chunk(ci, ub, ib, f):
        def group_body(g, carry2, f=f):
            for jj in range(16):
                e = g * 16 + jj
                prods = []
                for c in range(D // 16):
                    u = ub[e, pl.ds(c * 16, 16)]
                    v = ib[e, pl.ds(c * 16, 16)]
                    prods.append(u * v)
                while len(prods) > 1:
                    prods = [a + b for a, b in zip(prods[::2], prods[1::2])]
                tbuf[jj, pl.ds(0, 16)] = prods[0]
            csums = [plsc.load_gather(
                tbuf, [iota16, jnp.full((16,), c, _i32)])
                for c in range(16)]
            while len(csums) > 1:
                csums = [a + b for a, b in zip(csums[::2], csums[1::2])]
            tot = csums[0]
            start = ci * CH + g * 16
            tot = (tot + ubias[f][pl.ds(start, 16)]
                   + ibias[f][pl.ds(start, 16)])
            if GB[f] != 0.0:
                tot = tot + GB[f]
            dots[f][pl.ds(start, 16)] = tot
            return carry2

        lax.fori_loop(0, CH // 16, group_body, 0)

    issue(0, 0, 0)
    for m in misc:
        m.wait()

    for f in range(5):
        def pair_body(s, carry, f=f):
            c0 = s * 2
            issue(f, c0 + 1, 1)
            wait(f, c0, 0)
            compute_chunk(c0, ubuf[0], ibuf[0], f)

            @pl.when(c0 + 2 < NCHUNK)
            def _():
                issue(f, c0 + 2, 0)

            wait(f, c0 + 1, 1)
            compute_chunk(c0 + 1, ubuf[1], ibuf[1], f)
            return carry

        lax.fori_loop(0, NCHUNK // 2, pair_body, 0)
        if f < 4:
            issue(f + 1, 0, 0)

    outs = (alpha_o, beta_o, lamda_o, gamma_o, delta_o)
    for f in range(5):
        pltpu.sync_copy(dots[f], outs[f].at[pl.ds(base, BPW)])
    pltpu.sync_copy(pricebuf, price_o.at[pl.ds(base, BPW)])
    pltpu.sync_copy(refbuf, ref_o.at[pl.ds(base, BPW)])
    d_outs = (d0_o, d1_o, d2_o, d3_o, d4_o)
    for r in range(5):
        pltpu.sync_copy(distT[r], d_outs[r].at[pl.ds(base, BPW)])


def _pt_body(al_r, be_r, la_r, ga_r, de_r, pr_r, rf_r,
             p0_r, p1_r, p2_r, p3_r, p4_r, o_r):
    alpha = al_r[...]
    beta = be_r[...]
    lamda = la_r[...]
    gamma = ga_r[...]
    delta = de_r[...]
    price = pr_r[...]
    refv = rf_r[...]
    ps = (p0_r[...], p1_r[...], p2_r[...], p3_r[...], p4_r[...])

    acc = jnp.zeros_like(alpha)
    for r in range(5):
        x = jnp.tanh((r + 1.0) - refv)
        pos = (x > 0).astype(_f32)
        neg = 1.0 - pos
        x_ = price * jnp.abs(x) + 1e-8
        v_exp = alpha * pos + beta * neg
        v = x_ ** v_exp
        value = v * (pos - lamda * neg)
        p = ps[r]
        one_m_p = 1.0 - p
        w_g = (p ** gamma) / ((p ** gamma + one_m_p ** gamma) ** (1.0 / gamma))
        w_d = (p ** delta) / ((p ** delta + one_m_p ** delta) ** (1.0 / delta))
        weight = pos * w_g + neg * w_d
        acc = acc + weight * value
    o_r[...] = acc


_pt_call = pl.pallas_call(
    _pt_body, out_shape=jax.ShapeDtypeStruct((B,), _f32))


def kernel(users, items, distribution, item_price, reference_point,
           uB_a, iB_a, uE_a, iE_a,
           uB_b, iB_b, uE_b, iE_b,
           uB_l, iB_l, uE_l, iE_l,
           uB_g, iB_g, uE_g, iE_g,
           uB_d, iB_d, uE_d, iE_d):
    users = users.astype(_i32)
    items = items.astype(_i32)
    ubcat = jnp.concatenate(
        [t.reshape(-1) for t in (uB_a, uB_b, uB_l, uB_g, uB_d,
                                 reference_point)])
    ibcat = jnp.concatenate(
        [t.reshape(-1) for t in (iB_a, iB_b, iB_l, iB_g, iB_d)]
        + [item_price])
    dist_cm = distribution.T.reshape(-1)

    (alpha, beta, lamda, gamma, delta, price_g, ref_g,
     p0, p1, p2, p3, p4) = \
        _sc_gather_dot(users, items, dist_cm, ubcat, ibcat,
                       uE_a, iE_a, uE_b, iE_b, uE_l, iE_l,
                       uE_g, iE_g, uE_d, iE_d)

    return _pt_call(alpha, beta, lamda, gamma, delta,
                    price_g, ref_g, p0, p1, p2, p3, p4)
